# baseline (speedup 1.0000x reference)
import numpy as np
import jax
import jax.numpy as jnp
from jax import lax
from jax.experimental import pallas as pl
from jax.experimental.pallas import tpu as pltpu
from jax.experimental.pallas import tpu_sc as plsc

NSLICE = 6
H = W = 160
HW = H * W
SB = NSLICE * HW
TRASH = 179200
SBP = 179456
BIG = 524288
NC, NS = 2, 16
ALPHAS = [float(a) for a in np.linspace(0.0, 1.0, NSLICE + 1, dtype=np.float32)]


def _minmax_call(f3, B, N):
    CH = 32768
    NCHUNK = N // CH

    def body(f_ref, lo_ref, hi_ref):
        c = pl.program_id(1)
        f = f_ref[...]
        lane = lax.broadcasted_iota(jnp.int32, f.shape, 2)
        m = (lane % 3) == 2
        lo = jnp.min(jnp.where(m, f, jnp.inf)).reshape(1, 1, 1)
        hi = jnp.max(jnp.where(m, f, -jnp.inf)).reshape(1, 1, 1)

        @pl.when(c == 0)
        def _():
            lo_ref[...] = lo
            hi_ref[...] = hi

        @pl.when(c != 0)
        def _():
            lo_ref[...] = jnp.minimum(lo_ref[...], lo)
            hi_ref[...] = jnp.maximum(hi_ref[...], hi)

    return pl.pallas_call(
        body,
        grid=(B, NCHUNK),
        in_specs=[pl.BlockSpec((1, 1, 3 * CH),
                               lambda b, c: (b, 0, c))],
        out_specs=[pl.BlockSpec((1, 1, 1), lambda b, c: (b, 0, 0)),
                   pl.BlockSpec((1, 1, 1), lambda b, c: (b, 0, 0))],
        out_shape=[jax.ShapeDtypeStruct((B, 1, 1), jnp.float32),
                   jax.ShapeDtypeStruct((B, 1, 1), jnp.float32)],
    )(f3)


def _weight_call(f, zlo, zhi, B, N):
    CH = 32768
    NCHUNK = N // CH
    f_spec = pl.BlockSpec((1, 1, 3 * CH), lambda b, c: (b * NCHUNK + c, 0, 0))
    scalar_spec = pl.BlockSpec((1, 1, 1), lambda b, c: (b, 0, 0))

    def body(f_ref, lo_ref, hi_ref, a_ref):
        fv = f_ref[...]
        lo = lo_ref[...]
        hi = hi_ref[...]
        lane = lax.broadcasted_iota(jnp.int32, fv.shape, 2)
        cmod = lane % 3
        g = (fv - (-1.0)) / 2.000001 * (W - 1)
        bad = (g < 0.0) | (g >= float(W))
        gi = jnp.clip(jnp.floor(g), 0.0, float(W - 1))
        xterm = gi + jnp.where(bad, float(BIG), 0.0)
        yterm = gi * float(W) + jnp.where(bad, float(BIG), 0.0)
        s = jnp.zeros_like(fv)
        for j in range(1, NSLICE + 1):
            e = lo + (hi - lo) * ALPHAS[j]
            s += jnp.where(fv >= e, float(HW), 0.0)
        a_ref[...] = jnp.where(cmod == 0, xterm,
                               jnp.where(cmod == 1, yterm, s))

    return pl.pallas_call(
        body,
        grid=(B, NCHUNK),
        in_specs=[f_spec, scalar_spec, scalar_spec],
        out_specs=f_spec,
        out_shape=jax.ShapeDtypeStruct((B * NCHUNK, 1, 3 * CH), jnp.float32),
    )(f.reshape(B * NCHUNK, 1, 3 * CH), zlo, zhi)


def _compact_call(a2, q, rows):
    RB = 1024

    def body(a_ref, q_ref, idx_ref):
        prod = jax.lax.dot_general(
            a_ref[...], q_ref[...], (((1,), (0,)), ((), ())),
            precision=jax.lax.Precision.HIGHEST,
            preferred_element_type=jnp.float32)
        idx_ref[...] = jnp.minimum(prod, float(TRASH)).astype(jnp.int32)

    return pl.pallas_call(
        body,
        grid=(rows // RB,),
        in_specs=[pl.BlockSpec((RB, 384), lambda r: (r, 0)),
                  pl.BlockSpec((384, 128), lambda r: (0, 0))],
        out_specs=pl.BlockSpec((RB, 128), lambda r: (r, 0)),
        out_shape=jax.ShapeDtypeStruct((rows, 128), jnp.int32),
    )(a2, q)


def _scatter_call(idx_flat, B, N):
    BPC = B // NC
    PPT = N // NS
    SHARE = SB // NS
    ZSHARE = SBP // NS

    mesh = plsc.VectorSubcoreMesh(core_axis_name="c", subcore_axis_name="s")

    def body(idx_hbm, out_hbm, idx_v, ones_v, zero_v, hist):
        cid = lax.axis_index("c")
        sid = lax.axis_index("s")

        def fill_ones(i, carry):
            ones_v[pl.ds(i * 16, 16)] = jnp.full((16,), 1.0, jnp.float32)
            return carry

        def fill_zero(i, carry):
            zero_v[pl.ds(i * 16, 16)] = jnp.zeros((16,), jnp.float32)
            return carry

        lax.fori_loop(0, PPT // 16, fill_ones, 0)
        lax.fori_loop(0, ZSHARE // 16, fill_zero, 0)

        for b in range(BPC):
            batch = cid * BPC + b
            pltpu.sync_copy(idx_hbm.at[pl.ds(batch * N + sid * PPT, PPT)],
                            idx_v)
            pltpu.sync_copy(zero_v, hist.at[pl.ds(sid * ZSHARE, ZSHARE)])
            plsc.subcore_barrier()
            pltpu.sync_copy(ones_v, hist.at[idx_v], add=True)
            plsc.subcore_barrier()
            pltpu.sync_copy(hist.at[pl.ds(sid * SHARE, SHARE)],
                            out_hbm.at[pl.ds(batch * SB + sid * SHARE,
                                             SHARE)])
            plsc.subcore_barrier()

    f = pl.kernel(
        body,
        out_type=jax.ShapeDtypeStruct((B * SB,), jnp.float32),
        mesh=mesh,
        scratch_types=[
            pltpu.VMEM((PPT,), jnp.int32),
            pltpu.VMEM((PPT,), jnp.float32),
            pltpu.VMEM((ZSHARE,), jnp.float32),
            pltpu.VMEM_SHARED((SBP,), jnp.float32),
        ],
    )
    return f(idx_flat)


def _normalize_call(counts, B):
    def body(c_ref, o_ref):
        bev = jnp.log1p(c_ref[...])
        bmin = jnp.min(bev)
        bmax = jnp.max(bev)
        o_ref[...] = (bev - bmin) / (bmax - bmin + 1e-6)

    return pl.pallas_call(
        body,
        grid=(B * NSLICE,),
        in_specs=[pl.BlockSpec((1, 1, HW), lambda i: (i, 0, 0))],
        out_specs=pl.BlockSpec((1, 1, HW), lambda i: (i, 0, 0)),
        out_shape=jax.ShapeDtypeStruct((B * NSLICE, 1, HW), jnp.float32),
    )(counts)


_QNP = np.zeros((384, 128), np.float32)
for _l in range(128):
    _QNP[3 * _l, _l] = 1.0
    _QNP[3 * _l + 1, _l] = 1.0
    _QNP[3 * _l + 2, _l] = 1.0


def kernel(xyz):
    B, N, _ = xyz.shape
    f = xyz.reshape(B, 1, 3 * N)
    zlo, zhi = _minmax_call(f, B, N)
    a = _weight_call(f, zlo, zhi, B, N)
    rows = B * N // 128
    idx = _compact_call(a.reshape(rows, 384), jnp.asarray(_QNP), rows)
    counts = _scatter_call(idx.reshape(B * N), B, N)
    bev = _normalize_call(counts.reshape(B * NSLICE, 1, HW), B)
    return bev.reshape(B, NSLICE, H, W)


# speedup vs baseline: 51.7630x; 51.7630x over previous
import numpy as np
import jax
import jax.numpy as jnp
from jax import lax
from jax.experimental import pallas as pl
from jax.experimental.pallas import tpu as pltpu
from jax.experimental.pallas import tpu_sc as plsc

NSLICE = 6
H = W = 160
HW = H * W
SB = NSLICE * HW
SBP = 153856
TRASH = SB
NC, NS = 2, 16
ALPHAS = [float(a) for a in np.linspace(0.0, 1.0, NSLICE + 1, dtype=np.float32)]

CH = 32768
CHR = CH // 128


def _minmax_call(z, B, N):
    NCHUNK = N // CH

    def body(z_ref, lo_ref, hi_ref):
        c = pl.program_id(0)
        zv = z_ref[...]
        lo = jnp.min(zv, axis=(1, 2), keepdims=True)
        hi = jnp.max(zv, axis=(1, 2), keepdims=True)

        @pl.when(c == 0)
        def _():
            lo_ref[...] = lo
            hi_ref[...] = hi

        @pl.when(c != 0)
        def _():
            lo_ref[...] = jnp.minimum(lo_ref[...], lo)
            hi_ref[...] = jnp.maximum(hi_ref[...], hi)

    return pl.pallas_call(
        body,
        grid=(NCHUNK,),
        in_specs=[pl.BlockSpec((B, CHR, 128), lambda c: (0, c, 0))],
        out_specs=[pl.BlockSpec((B, 1, 1), lambda c: (0, 0, 0)),
                   pl.BlockSpec((B, 1, 1), lambda c: (0, 0, 0))],
        out_shape=[jax.ShapeDtypeStruct((B, 1, 1), jnp.float32),
                   jax.ShapeDtypeStruct((B, 1, 1), jnp.float32)],
    )(z.reshape(B, N // 128, 128))


def _index_call(x, y, z, zlo, zhi, B, N):
    NCHUNK = N // CH
    chunk_spec = pl.BlockSpec((B, CHR, 128), lambda c: (0, c, 0))
    scalar_spec = pl.BlockSpec((B, 1, 1), lambda c: (0, 0, 0))

    def body(x_ref, y_ref, z_ref, lo_ref, hi_ref, idx_ref):
        xv = x_ref[...]
        yv = y_ref[...]
        zv = z_ref[...]
        lo = lo_ref[...]
        hi = hi_ref[...]
        gx = (xv - (-1.0)) / 2.000001 * (W - 1)
        gy = (yv - (-1.0)) / 2.000001 * (H - 1)
        valid = (gy >= 0.0) & (gy < H) & (gx >= 0.0) & (gx < W)
        iy = jnp.clip(gy.astype(jnp.int32), 0, H - 1)
        ix = jnp.clip(gx.astype(jnp.int32), 0, W - 1)
        flat = iy * W + ix
        s = jnp.zeros_like(flat)
        for j in range(1, NSLICE + 1):
            e = lo + (hi - lo) * ALPHAS[j]
            s += (zv >= e).astype(jnp.int32)
        idx_ref[...] = jnp.where(valid & (s < NSLICE), s * HW + flat, TRASH)

    return pl.pallas_call(
        body,
        grid=(NCHUNK,),
        in_specs=[chunk_spec, chunk_spec, chunk_spec, scalar_spec,
                  scalar_spec],
        out_specs=chunk_spec,
        out_shape=jax.ShapeDtypeStruct((B, N // 128, 128), jnp.int32),
    )(x.reshape(B, N // 128, 128), y.reshape(B, N // 128, 128),
      z.reshape(B, N // 128, 128), zlo, zhi)


def _scatter_call(idx_flat, B, N):
    BPC = B // NC
    PPT = N // NS
    SHARE = SB // NS
    ZSHARE = SBP // NS

    mesh = plsc.VectorSubcoreMesh(core_axis_name="c", subcore_axis_name="s")

    def body(idx_hbm, out_hbm, idx_v, ones_v, zero_v, hist):
        cid = lax.axis_index("c")
        sid = lax.axis_index("s")

        def fill_ones(i, carry):
            for j in range(8):
                ones_v[pl.ds(i * 128 + j * 16, 16)] = jnp.full(
                    (16,), 1.0, jnp.float32)
            return carry

        def fill_zero(i, carry):
            for j in range(8):
                zero_v[pl.ds(i * 128 + j * 16, 16)] = jnp.zeros(
                    (16,), jnp.float32)
            return carry

        lax.fori_loop(0, PPT // 128, fill_ones, 0)
        lax.fori_loop(0, ZSHARE // 128, fill_zero, 0)

        for b in range(BPC):
            batch = cid * BPC + b
            pltpu.sync_copy(idx_hbm.at[pl.ds(batch * N + sid * PPT, PPT)],
                            idx_v)
            pltpu.sync_copy(zero_v, hist.at[pl.ds(sid * ZSHARE, ZSHARE)])
            plsc.subcore_barrier()
            pltpu.sync_copy(ones_v, hist.at[idx_v], add=True)
            plsc.subcore_barrier()
            pltpu.sync_copy(hist.at[pl.ds(sid * SHARE, SHARE)],
                            out_hbm.at[pl.ds(batch * SB + sid * SHARE,
                                             SHARE)])
            plsc.subcore_barrier()

    f = pl.kernel(
        body,
        out_type=jax.ShapeDtypeStruct((B * SB,), jnp.float32),
        mesh=mesh,
        scratch_types=[
            pltpu.VMEM((PPT,), jnp.int32),
            pltpu.VMEM((PPT,), jnp.float32),
            pltpu.VMEM((ZSHARE,), jnp.float32),
            pltpu.VMEM_SHARED((SBP,), jnp.float32),
        ],
    )
    return f(idx_flat)


def _normalize_call(counts, B):
    ROWS = HW // 128

    def body(c_ref, o_ref):
        bev = jnp.log1p(c_ref[...])
        bmin = jnp.min(bev)
        bmax = jnp.max(bev)
        o_ref[...] = (bev - bmin) / (bmax - bmin + 1e-6)

    return pl.pallas_call(
        body,
        grid=(B * NSLICE,),
        in_specs=[pl.BlockSpec((ROWS, 128), lambda i: (i, 0))],
        out_specs=pl.BlockSpec((ROWS, 128), lambda i: (i, 0)),
        out_shape=jax.ShapeDtypeStruct((B * NSLICE * ROWS, 128),
                                       jnp.float32),
    )(counts)


def kernel(xyz):
    B, N, _ = xyz.shape
    x = xyz[..., 0]
    y = xyz[..., 1]
    z = xyz[..., 2]
    zlo, zhi = _minmax_call(z, B, N)
    idx = _index_call(x, y, z, zlo, zhi, B, N)
    counts = _scatter_call(idx.reshape(B * N), B, N)
    bev = _normalize_call(counts.reshape(B * NSLICE * (HW // 128), 128), B)
    return bev.reshape(B, NSLICE, H, W)


# speedup vs baseline: 57.0968x; 1.1030x over previous
import numpy as np
import jax
import jax.numpy as jnp
from jax import lax
from jax.experimental import pallas as pl
from jax.experimental.pallas import tpu as pltpu
from jax.experimental.pallas import tpu_sc as plsc

NSLICE = 6
H = W = 160
HW = H * W
SB = NSLICE * HW
SBP = 153856
TRASH = SB
NC, NS = 2, 16
ALPHAS = [float(a) for a in np.linspace(0.0, 1.0, NSLICE + 1, dtype=np.float32)]

CH = 32768
CHR = CH // 128


def _minmax_call(z, B, N):
    NCHUNK = N // CH

    def body(z_ref, lo_ref, hi_ref):
        c = pl.program_id(0)
        zv = z_ref[...]
        lo = jnp.min(zv, axis=(1, 2), keepdims=True)
        hi = jnp.max(zv, axis=(1, 2), keepdims=True)

        @pl.when(c == 0)
        def _():
            lo_ref[...] = lo
            hi_ref[...] = hi

        @pl.when(c != 0)
        def _():
            lo_ref[...] = jnp.minimum(lo_ref[...], lo)
            hi_ref[...] = jnp.maximum(hi_ref[...], hi)

    return pl.pallas_call(
        body,
        grid=(NCHUNK,),
        in_specs=[pl.BlockSpec((B, CHR, 128), lambda c: (0, c, 0))],
        out_specs=[pl.BlockSpec((B, 1, 1), lambda c: (0, 0, 0)),
                   pl.BlockSpec((B, 1, 1), lambda c: (0, 0, 0))],
        out_shape=[jax.ShapeDtypeStruct((B, 1, 1), jnp.float32),
                   jax.ShapeDtypeStruct((B, 1, 1), jnp.float32)],
    )(z.reshape(B, N // 128, 128))


def _index_call(x, y, z, zlo, zhi, B, N, g):
    NCHUNK = N // CH
    BH = B // 2
    chunk_spec = pl.BlockSpec((BH, CHR, 128), lambda c: (g, c, 0))
    scalar_spec = pl.BlockSpec((BH, 1, 1), lambda c: (g, 0, 0))

    def body(x_ref, y_ref, z_ref, lo_ref, hi_ref, idx_ref):
        xv = x_ref[...]
        yv = y_ref[...]
        zv = z_ref[...]
        lo = lo_ref[...]
        hi = hi_ref[...]
        gx = (xv - (-1.0)) / 2.000001 * (W - 1)
        gy = (yv - (-1.0)) / 2.000001 * (H - 1)
        valid = (gy >= 0.0) & (gy < H) & (gx >= 0.0) & (gx < W)
        iy = jnp.clip(gy.astype(jnp.int32), 0, H - 1)
        ix = jnp.clip(gx.astype(jnp.int32), 0, W - 1)
        flat = iy * W + ix
        s = jnp.zeros_like(flat)
        for j in range(1, NSLICE + 1):
            e = lo + (hi - lo) * ALPHAS[j]
            s += (zv >= e).astype(jnp.int32)
        idx_ref[...] = jnp.where(valid & (s < NSLICE), s * HW + flat, TRASH)

    out_spec = pl.BlockSpec((BH, CHR, 128), lambda c: (0, c, 0))
    return pl.pallas_call(
        body,
        grid=(NCHUNK,),
        in_specs=[chunk_spec, chunk_spec, chunk_spec, scalar_spec,
                  scalar_spec],
        out_specs=out_spec,
        out_shape=jax.ShapeDtypeStruct((BH, N // 128, 128), jnp.int32),
    )(x.reshape(B, N // 128, 128), y.reshape(B, N // 128, 128),
      z.reshape(B, N // 128, 128), zlo, zhi)


def _scatter_call(idx_flat, B, N):
    BPC = B // NC
    PPT = N // NS
    SHARE = SB // NS
    ZSHARE = SBP // NS

    mesh = plsc.VectorSubcoreMesh(core_axis_name="c", subcore_axis_name="s")

    def body(idx_hbm, out_hbm, idx_v, ones_v, zero_v, hist):
        cid = lax.axis_index("c")
        sid = lax.axis_index("s")

        def fill_ones(i, carry):
            for j in range(8):
                ones_v[pl.ds(i * 128 + j * 16, 16)] = jnp.full(
                    (16,), 1.0, jnp.float32)
            return carry

        def fill_zero(i, carry):
            for j in range(8):
                zero_v[pl.ds(i * 128 + j * 16, 16)] = jnp.zeros(
                    (16,), jnp.float32)
            return carry

        lax.fori_loop(0, PPT // 128, fill_ones, 0)
        lax.fori_loop(0, ZSHARE // 128, fill_zero, 0)

        for b in range(BPC):
            batch = cid * BPC + b
            pltpu.sync_copy(idx_hbm.at[pl.ds(batch * N + sid * PPT, PPT)],
                            idx_v)
            pltpu.sync_copy(zero_v, hist.at[pl.ds(sid * ZSHARE, ZSHARE)])
            plsc.subcore_barrier()
            pltpu.sync_copy(ones_v, hist.at[idx_v], add=True)
            plsc.subcore_barrier()
            pltpu.sync_copy(hist.at[pl.ds(sid * SHARE, SHARE)],
                            out_hbm.at[pl.ds(batch * SB + sid * SHARE,
                                             SHARE)])
            plsc.subcore_barrier()

    f = pl.kernel(
        body,
        out_type=jax.ShapeDtypeStruct((B * SB,), jnp.float32),
        mesh=mesh,
        scratch_types=[
            pltpu.VMEM((PPT,), jnp.int32),
            pltpu.VMEM((PPT,), jnp.float32),
            pltpu.VMEM((ZSHARE,), jnp.float32),
            pltpu.VMEM_SHARED((SBP,), jnp.float32),
        ],
    )
    return f(idx_flat)


def _normalize_call(counts, B):
    ROWS = HW // 128

    def body(c_ref, o_ref):
        bev = jnp.log1p(c_ref[...])
        bmin = jnp.min(bev)
        bmax = jnp.max(bev)
        o_ref[...] = (bev - bmin) / (bmax - bmin + 1e-6)

    return pl.pallas_call(
        body,
        grid=(B * NSLICE,),
        in_specs=[pl.BlockSpec((ROWS, 128), lambda i: (i, 0))],
        out_specs=pl.BlockSpec((ROWS, 128), lambda i: (i, 0)),
        out_shape=jax.ShapeDtypeStruct((B * NSLICE * ROWS, 128),
                                       jnp.float32),
    )(counts)


def kernel(xyz):
    B, N, _ = xyz.shape
    BH = B // 2
    x = xyz[..., 0]
    y = xyz[..., 1]
    z = xyz[..., 2]
    zlo, zhi = _minmax_call(z, B, N)
    idx0 = _index_call(x, y, z, zlo, zhi, B, N, 0)
    counts0 = _scatter_call(idx0.reshape(BH * N), BH, N)
    idx1 = _index_call(x, y, z, zlo, zhi, B, N, 1)
    counts1 = _scatter_call(idx1.reshape(BH * N), BH, N)
    bev0 = _normalize_call(
        counts0.reshape(BH * NSLICE * (HW // 128), 128), BH)
    bev1 = _normalize_call(
        counts1.reshape(BH * NSLICE * (HW // 128), 128), BH)
    return jnp.concatenate(
        [bev0.reshape(BH, NSLICE, H, W), bev1.reshape(BH, NSLICE, H, W)],
        axis=0)
